# Initial kernel scaffold; baseline (speedup 1.0000x reference)
import jax
import jax.numpy as jnp
from jax.experimental import pallas as pl


def kernel(path):
    raise NotImplementedError("write your pallas kernel here")


# speedup vs baseline: 26.0579x; 26.0579x over previous
import jax
import jax.numpy as jnp
from jax import lax
from jax.experimental import pallas as pl
from jax.experimental.pallas import tpu as pltpu

_D = 5
_S = _D + _D**2 + _D**3 + _D**4
_C = 4.0


def _outer(a, v):
    A = a.shape[0]
    return (a[:, None] * v[None]).reshape(A * _D, a.shape[1], a.shape[2])


def _esig_kernel(x_ref, o_ref):
    L = x_ref.shape[0]
    SB, LN = x_ref.shape[3], x_ref.shape[4]
    f32 = jnp.float32

    x0 = x_ref[0, :, 0]

    def zeros(n):
        return jnp.zeros((n, SB, LN), f32)

    def step(t, carry):
        c1, c2, c3, c4, xp = carry
        xt = x_ref[t, :, 0]
        v = xt - xp
        v2 = v * 0.5
        v3 = v * (1.0 / 3.0)
        v4 = v * 0.25
        n4 = c4 + _outer(c3 + _outer(c2 + _outer(c1 + v4, v3), v2), v)
        n3 = c3 + _outer(c2 + _outer(c1 + v3, v2), v)
        n2 = c2 + _outer(c1 + v2, v)
        n1 = c1 + v
        return (n1, n2, n3, n4, xt)

    init = (zeros(_D), zeros(_D**2), zeros(_D**3), zeros(_D**4), x0)
    c1, c2, c3, c4, _ = lax.fori_loop(1, L, step, init)

    s1 = jnp.sum(c1 * c1, axis=0)
    s2 = jnp.sum(c2 * c2, axis=0)
    s3 = jnp.sum(c3 * c3, axis=0)
    s4 = jnp.sum(c4 * c4, axis=0)
    normquad = 1.0 + s1 + s2 + s3 + s4
    phi = jnp.where(
        normquad > _C, _C + _C * _C * (1.0 / _C - 1.0 / normquad), normquad
    )
    c0 = 1.0 - phi

    def bis(_, lh):
        lo, hi = lh
        mid = 0.5 * (lo + hi)
        y = mid * mid
        fv = c0 + y * (s1 + y * (s2 + y * (s3 + y * s4)))
        pos = fv > 0
        return (jnp.where(pos, lo, mid), jnp.where(pos, mid, hi))

    lo0 = jnp.zeros((SB, LN), f32)
    hi0 = jnp.full((SB, LN), 2.0, f32)
    lo, hi = lax.fori_loop(0, 60, bis, (lo0, hi0))
    lam = jnp.minimum(0.5 * (lo + hi), 1.0)
    lam2 = lam * lam
    lam3 = lam2 * lam
    lam4 = lam2 * lam2

    r1 = jnp.sum(c1 * lam[None], axis=-1)
    r2 = jnp.sum(c2 * lam2[None], axis=-1)
    r3 = jnp.sum(c3 * lam3[None], axis=-1)
    r4 = jnp.sum(c4 * lam4[None], axis=-1)
    o_ref[0] = jnp.concatenate([r1, r2, r3, r4], axis=0)


def kernel(path):
    B, N, L, D = path.shape
    P = B * N
    SB, LN = 8, 128
    PB = SB * LN
    G = P // PB

    x = path.reshape(P, L, D).transpose(1, 2, 0).reshape(L, D, G, SB, LN)

    out = pl.pallas_call(
        _esig_kernel,
        grid=(G,),
        in_specs=[pl.BlockSpec((L, D, 1, SB, LN), lambda i: (0, 0, i, 0, 0))],
        out_specs=pl.BlockSpec((1, _S, SB), lambda i: (i, 0, 0)),
        out_shape=jax.ShapeDtypeStruct((G, _S, SB), jnp.float32),
        compiler_params=pltpu.CompilerParams(
            dimension_semantics=("parallel",),
            vmem_limit_bytes=56 * 1024 * 1024,
        ),
    )(x)

    o = out.reshape(G, _S, SB // 2, 2).sum(axis=-1)
    o = o.transpose(0, 2, 1).reshape(B, _S) * (1.0 / N)
    return o
